# Initial kernel scaffold; baseline (speedup 1.0000x reference)
import jax
import jax.numpy as jnp
from jax.experimental import pallas as pl


def kernel(x):
    raise NotImplementedError("write your pallas kernel here")


# speedup vs baseline: 1.3137x; 1.3137x over previous
import jax
import jax.numpy as jnp
from jax import lax
from jax.experimental import pallas as pl
from jax.experimental.pallas import tpu as pltpu

R, C = 128, 32768
BC = 4096
NB = C // BC
INT_MAX = 2147483647


def _amax_body(x_ref, idx_ref, m_scr, i_scr):
    j = pl.program_id(0)
    blk = x_ref[...]
    m = jnp.max(blk, axis=1, keepdims=True)
    liota = lax.broadcasted_iota(jnp.int32, blk.shape, 1) + j * BC
    cand = jnp.where(blk == m, liota, INT_MAX)
    ci = jnp.min(cand, axis=1, keepdims=True)

    @pl.when(j == 0)
    def _():
        m_scr[...] = m
        i_scr[...] = ci

    @pl.when(j > 0)
    def _():
        upd = m > m_scr[...]
        i_scr[...] = jnp.where(upd, ci, i_scr[...])
        m_scr[...] = jnp.where(upd, m, m_scr[...])

    @pl.when(j == NB - 1)
    def _():
        idx_ref[...] = i_scr[...]


def _onehot_body(idx_ref, out_ref):
    j = pl.program_id(0)
    liota = lax.broadcasted_iota(jnp.int32, (R, BC), 1) + j * BC
    out_ref[...] = jnp.where(liota == idx_ref[...], 1.0, 0.0).astype(jnp.float32)


def kernel(x):
    idx = pl.pallas_call(
        _amax_body,
        grid=(NB,),
        in_specs=[pl.BlockSpec((R, BC), lambda j: (0, j))],
        out_specs=pl.BlockSpec((R, 1), lambda j: (0, 0)),
        out_shape=jax.ShapeDtypeStruct((R, 1), jnp.int32),
        scratch_shapes=[
            pltpu.VMEM((R, 1), jnp.float32),
            pltpu.VMEM((R, 1), jnp.int32),
        ],
    )(x)
    out = pl.pallas_call(
        _onehot_body,
        grid=(NB,),
        in_specs=[pl.BlockSpec((R, 1), lambda j: (0, 0))],
        out_specs=pl.BlockSpec((R, BC), lambda j: (0, j)),
        out_shape=jax.ShapeDtypeStruct((R, C), jnp.float32),
    )(idx)
    return out
